# Initial kernel scaffold; baseline (speedup 1.0000x reference)
import functools

import jax
import jax.numpy as jnp
from jax import lax
from jax.experimental import pallas as pl
from jax.experimental.pallas import tpu as pltpu
from jax.experimental.pallas import tpu_sc as plsc

VOCAB = 1_000_000
EMB = 32
N_CLS = 5
PROJ = 16
BATCH = 16384
SEQ = 200
HALF = SEQ // 2
NC, NS = 2, 16
NW = NC * NS
ROWS_PER_W = BATCH // NW
CHUNK = ROWS_PER_W // 2
RING = 4
PROJ_BLK = 8000


def _proj_body(emb_ref, w_ref, out_ref):
    out_ref[...] = jnp.dot(emb_ref[...], w_ref[...],
                           preferred_element_type=jnp.float32)


def _project_table(emb_table, w_pad):
    return pl.pallas_call(
        _proj_body,
        grid=(VOCAB // PROJ_BLK,),
        in_specs=[
            pl.BlockSpec((PROJ_BLK, EMB), lambda i: (i, 0)),
            pl.BlockSpec((EMB, PROJ), lambda i: (0, 0)),
        ],
        out_specs=pl.BlockSpec((PROJ_BLK, PROJ), lambda i: (i, 0)),
        out_shape=jax.ShapeDtypeStruct((VOCAB, PROJ), jnp.float32),
    )(emb_table, w_pad)


_mesh = plsc.VectorSubcoreMesh(core_axis_name="c", subcore_axis_name="s")


@functools.partial(
    pl.kernel,
    out_type=jax.ShapeDtypeStruct((BATCH, PROJ), jnp.float32),
    mesh=_mesh,
    scratch_types=[
        pltpu.VMEM((CHUNK, 2, HALF), jnp.int32),
        pltpu.VMEM((RING, 2, HALF, PROJ), jnp.float32),
        pltpu.VMEM((ROWS_PER_W, PROJ), jnp.float32),
        pltpu.VMEM((PROJ,), jnp.float32),
        pltpu.SemaphoreType.DMA,
        pltpu.SemaphoreType.DMA,
        pltpu.SemaphoreType.DMA,
        pltpu.SemaphoreType.DMA,
    ],
)
def _sc_gather(tab_hbm, x_hbm, bias_hbm, out_hbm,
               idx_v, rows_v, out_v, bias_v, s0, s1, s2, s3):
    sems = (s0, s1, s2, s3)
    wid = lax.axis_index("s") * NC + lax.axis_index("c")
    base = wid * ROWS_PER_W

    pltpu.sync_copy(bias_hbm, bias_v)
    bias = bias_v[...]
    inv = jnp.float32(1.0 / SEQ)

    def issue(row_in_chunk, slot):
        for h in range(2):
            pltpu.async_copy(tab_hbm.at[idx_v.at[row_in_chunk, h]],
                             rows_v.at[slot, h], sems[slot])

    def drain(slot):
        for h in range(2):
            pltpu.make_async_copy(tab_hbm.at[pl.ds(0, HALF)],
                                  rows_v.at[slot, h], sems[slot]).wait()

    for half in range(2):
        pltpu.sync_copy(x_hbm.at[pl.ds(base + half * CHUNK, CHUNK)], idx_v)
        for q in range(RING):
            issue(q, q)

        def body(r_outer, _, half=half):
            for q in range(RING):
                row = r_outer * RING + q
                drain(q)
                accs = [rows_v[q, 0, j] for j in range(4)]
                for h in range(2):
                    for j in range(HALF):
                        if h == 0 and j < 4:
                            continue
                        accs[j % 4] = accs[j % 4] + rows_v[q, h, j]
                acc = (accs[0] + accs[1]) + (accs[2] + accs[3])
                out_v[half * CHUNK + row] = acc * inv + bias

                @pl.when(row + RING < CHUNK)
                def _():
                    issue(row + RING, q)
            return _

        lax.fori_loop(0, CHUNK // RING, body, None)

    pltpu.sync_copy(out_v, out_hbm.at[pl.ds(base, ROWS_PER_W)])


def kernel(x, emb_table, fc_w, fc_b):
    w_pad = jnp.zeros((EMB, PROJ), jnp.float32).at[:, :N_CLS].set(fc_w.T)
    bias_pad = jnp.zeros((PROJ,), jnp.float32).at[:N_CLS].set(fc_b)
    tab = _project_table(emb_table, w_pad)
    x3 = x.reshape(BATCH, 2, HALF)
    out16 = _sc_gather(tab, x3, bias_pad)
    return out16[:, :N_CLS]


# speedup vs baseline: 9.8438x; 9.8438x over previous
import functools

import jax
import jax.numpy as jnp
from jax import lax
from jax.experimental import pallas as pl
from jax.experimental.pallas import tpu as pltpu
from jax.experimental.pallas import tpu_sc as plsc

VOCAB = 1_000_000
EMB = 32
N_CLS = 5
PROJ = 16
BATCH = 16384
SEQ = 200
HALF = SEQ // 2
NC, NS = 2, 16
NW = NC * NS
ROWS_PER_W = BATCH // NW
CHUNK = ROWS_PER_W // 2
RING = 4
PROJ_BLK = 8000


def _proj_body(emb_ref, w_ref, out_ref):
    out_ref[...] = jnp.dot(emb_ref[...], w_ref[...],
                           preferred_element_type=jnp.float32)


def _project_table(emb_table, w_pad):
    return pl.pallas_call(
        _proj_body,
        grid=(VOCAB // PROJ_BLK,),
        in_specs=[
            pl.BlockSpec((PROJ_BLK, EMB), lambda i: (i, 0)),
            pl.BlockSpec((EMB, PROJ), lambda i: (0, 0)),
        ],
        out_specs=pl.BlockSpec((PROJ_BLK, PROJ), lambda i: (i, 0)),
        out_shape=jax.ShapeDtypeStruct((VOCAB, PROJ), jnp.float32),
    )(emb_table, w_pad)


_mesh = plsc.VectorSubcoreMesh(core_axis_name="c", subcore_axis_name="s")


@functools.partial(
    pl.kernel,
    out_type=jax.ShapeDtypeStruct((BATCH, PROJ), jnp.float32),
    mesh=_mesh,
    scratch_types=[
        pltpu.VMEM((CHUNK, 2, HALF), jnp.int32),
        pltpu.VMEM((RING, 2, HALF, PROJ), jnp.float32),
        pltpu.VMEM((ROWS_PER_W, PROJ), jnp.float32),
        pltpu.VMEM((PROJ,), jnp.float32),
        pltpu.SemaphoreType.DMA,
        pltpu.SemaphoreType.DMA,
        pltpu.SemaphoreType.DMA,
        pltpu.SemaphoreType.DMA,
    ],
    compiler_params=pltpu.CompilerParams(use_tc_tiling_on_sc=False),
)
def _sc_gather(tab_hbm, x_hbm, bias_hbm, out_hbm,
               idx_v, rows_v, out_v, bias_v, s0, s1, s2, s3):
    sems = (s0, s1, s2, s3)
    wid = lax.axis_index("s") * NC + lax.axis_index("c")
    base = wid * ROWS_PER_W

    pltpu.sync_copy(bias_hbm, bias_v)
    bias = bias_v[...]
    inv = jnp.float32(1.0 / SEQ)

    def issue(row_in_chunk, slot):
        for h in range(2):
            pltpu.async_copy(tab_hbm.at[idx_v.at[row_in_chunk, h]],
                             rows_v.at[slot, h], sems[slot])

    def drain(row_in_chunk, slot):
        for h in range(2):
            pltpu.make_async_copy(tab_hbm.at[idx_v.at[row_in_chunk, h]],
                                  rows_v.at[slot, h], sems[slot]).wait()

    for half in range(2):
        pltpu.sync_copy(x_hbm.at[pl.ds(base + half * CHUNK, CHUNK)], idx_v)
        for q in range(RING):
            issue(q, q)

        def body(r_outer, carry, half=half):
            for q in range(RING):
                row = r_outer * RING + q
                drain(row, q)
                accs = [rows_v[q, 0, j] for j in range(4)]
                for h in range(2):
                    for j in range(HALF):
                        if h == 0 and j < 4:
                            continue
                        accs[j % 4] = accs[j % 4] + rows_v[q, h, j]
                acc = (accs[0] + accs[1]) + (accs[2] + accs[3])
                out_v[half * CHUNK + row] = acc * inv + bias

                @pl.when(row + RING < CHUNK)
                def _issue_next(row=row, q=q):
                    issue(row + RING, q)
            return carry

        lax.fori_loop(0, CHUNK // RING, body, None)

    pltpu.sync_copy(out_v, out_hbm.at[pl.ds(base, ROWS_PER_W)])


def kernel(x, emb_table, fc_w, fc_b):
    w_pad = jnp.zeros((EMB, PROJ), jnp.float32).at[:, :N_CLS].set(fc_w.T)
    bias_pad = jnp.zeros((PROJ,), jnp.float32).at[:N_CLS].set(fc_b)
    tab = _project_table(emb_table, w_pad)
    x3 = x.reshape(BATCH, 2, HALF)
    out16 = _sc_gather(tab, x3, bias_pad)
    return out16[:, :N_CLS]


# speedup vs baseline: 10.7996x; 1.0971x over previous
import functools

import jax
import jax.numpy as jnp
from jax import lax
from jax.experimental import pallas as pl
from jax.experimental.pallas import tpu as pltpu
from jax.experimental.pallas import tpu_sc as plsc

VOCAB = 1_000_000
EMB = 32
N_CLS = 5
NPAD = 128
BATCH = 16384
SEQ = 200
HALF = SEQ // 2
NC, NS = 2, 16
NW = NC * NS
ROWS_PER_W = BATCH // NW
CHUNK = ROWS_PER_W // 2
RING = 4
LIN_BLK = 2048


_mesh = plsc.VectorSubcoreMesh(core_axis_name="c", subcore_axis_name="s")


@functools.partial(
    pl.kernel,
    out_type=jax.ShapeDtypeStruct((BATCH, EMB), jnp.float32),
    mesh=_mesh,
    scratch_types=[
        pltpu.VMEM((CHUNK, 2, HALF), jnp.int32),
        pltpu.VMEM((RING, 2, HALF, EMB), jnp.float32),
        pltpu.VMEM((ROWS_PER_W, EMB), jnp.float32),
        pltpu.SemaphoreType.DMA,
        pltpu.SemaphoreType.DMA,
        pltpu.SemaphoreType.DMA,
        pltpu.SemaphoreType.DMA,
    ],
    compiler_params=pltpu.CompilerParams(use_tc_tiling_on_sc=False),
)
def _sc_pool(tab_hbm, x_hbm, out_hbm, idx_v, rows_v, out_v, s0, s1, s2, s3):
    sems = (s0, s1, s2, s3)
    wid = lax.axis_index("s") * NC + lax.axis_index("c")
    base = wid * ROWS_PER_W
    inv = jnp.float32(1.0 / SEQ)

    def issue(row_in_chunk, slot):
        for h in range(2):
            pltpu.async_copy(tab_hbm.at[idx_v.at[row_in_chunk, h]],
                             rows_v.at[slot, h], sems[slot])

    def drain(row_in_chunk, slot):
        for h in range(2):
            pltpu.make_async_copy(tab_hbm.at[idx_v.at[row_in_chunk, h]],
                                  rows_v.at[slot, h], sems[slot]).wait()

    for half in range(2):
        pltpu.sync_copy(x_hbm.at[pl.ds(base + half * CHUNK, CHUNK)], idx_v)
        for q in range(RING):
            issue(q, q)

        def body(r_outer, carry, half=half):
            for q in range(RING):
                row = r_outer * RING + q
                drain(row, q)
                acc = [[rows_v[q, 0, j, pl.ds(p * 16, 16)] for j in range(2)]
                       for p in range(2)]
                for h in range(2):
                    for j in range(HALF):
                        if h == 0 and j < 2:
                            continue
                        for p in range(2):
                            acc[p][j % 2] = (acc[p][j % 2]
                                             + rows_v[q, h, j, pl.ds(p * 16, 16)])
                out_v[half * CHUNK + row, pl.ds(0, 16)] = (acc[0][0] + acc[0][1]) * inv
                out_v[half * CHUNK + row, pl.ds(16, 16)] = (acc[1][0] + acc[1][1]) * inv

                @pl.when(row + RING < CHUNK)
                def _issue_next(row=row, q=q):
                    issue(row + RING, q)
            return carry

        lax.fori_loop(0, CHUNK // RING, body, None)

    pltpu.sync_copy(out_v, out_hbm.at[pl.ds(base, ROWS_PER_W)])


def _lin_body(pool_ref, w_ref, b_ref, out_ref):
    out_ref[...] = (jnp.dot(pool_ref[...], w_ref[...],
                            preferred_element_type=jnp.float32)
                    + b_ref[...])


def _linear(pooled, w_pad, b_pad):
    return pl.pallas_call(
        _lin_body,
        grid=(BATCH // LIN_BLK,),
        in_specs=[
            pl.BlockSpec((LIN_BLK, EMB), lambda i: (i, 0)),
            pl.BlockSpec((EMB, NPAD), lambda i: (0, 0)),
            pl.BlockSpec((1, NPAD), lambda i: (0, 0)),
        ],
        out_specs=pl.BlockSpec((LIN_BLK, NPAD), lambda i: (i, 0)),
        out_shape=jax.ShapeDtypeStruct((BATCH, NPAD), jnp.float32),
    )(pooled, w_pad, b_pad)


def kernel(x, emb_table, fc_w, fc_b):
    x3 = x.reshape(BATCH, 2, HALF)
    pooled = _sc_pool(emb_table, x3)
    w_pad = jnp.zeros((EMB, NPAD), jnp.float32).at[:, :N_CLS].set(fc_w.T)
    b_pad = jnp.zeros((1, NPAD), jnp.float32).at[0, :N_CLS].set(fc_b)
    out = _linear(pooled, w_pad, b_pad)
    return out[:, :N_CLS]


# speedup vs baseline: 32.2129x; 2.9828x over previous
import functools

import jax
import jax.numpy as jnp
from jax import lax
from jax.experimental import pallas as pl
from jax.experimental.pallas import tpu as pltpu
from jax.experimental.pallas import tpu_sc as plsc

VOCAB = 1_000_000
EMB = 32
N_CLS = 5
PROJ = 16
BATCH = 16384
SEQ = 200
HALF = SEQ // 2
NC, NS = 2, 16
NW = NC * NS
ROWS_PER_W = BATCH // NW
CHUNK = ROWS_PER_W // 2
RING = 4

NSLAB = 8
SLAB = 1 << 17
VPAD = NSLAB * SLAB
SBLK = 4096
NBLK = SLAB // SBLK
LAST_BLK = VOCAB // SBLK


def _pack_body(*refs):
    a_refs, w_ref, out_ref = refs[:NSLAB], refs[NSLAB], refs[NSLAB + 1]
    a8 = jnp.concatenate([r[...] for r in a_refs], axis=0)
    out_ref[...] = lax.dot_general(a8, w_ref[...],
                                   (((0,), (0,)), ((), ())),
                                   preferred_element_type=jnp.float32)


def _pack_table(emb_t, w8):
    in_specs = [
        pl.BlockSpec((EMB, SBLK),
                     lambda j, s=s: (0, jnp.minimum(s * NBLK + j, LAST_BLK)))
        for s in range(NSLAB)
    ] + [pl.BlockSpec((NSLAB * EMB, 128), lambda j: (0, 0))]
    return pl.pallas_call(
        _pack_body,
        grid=(NBLK,),
        in_specs=in_specs,
        out_specs=pl.BlockSpec((SBLK, 128), lambda j: (j, 0)),
        out_shape=jax.ShapeDtypeStruct((SLAB, 128), jnp.float32),
    )(*([emb_t] * NSLAB), w8)


_mesh = plsc.VectorSubcoreMesh(core_axis_name="c", subcore_axis_name="s")


@functools.partial(
    pl.kernel,
    out_type=jax.ShapeDtypeStruct((BATCH, PROJ), jnp.float32),
    mesh=_mesh,
    scratch_types=[
        pltpu.VMEM((CHUNK, 2, HALF), jnp.int32),
        pltpu.VMEM((RING, 2, HALF), jnp.int32),
        pltpu.VMEM((RING, 2, HALF, PROJ), jnp.float32),
        pltpu.VMEM((ROWS_PER_W, PROJ), jnp.float32),
        pltpu.VMEM((PROJ,), jnp.float32),
        pltpu.SemaphoreType.DMA,
        pltpu.SemaphoreType.DMA,
        pltpu.SemaphoreType.DMA,
        pltpu.SemaphoreType.DMA,
    ],
    compiler_params=pltpu.CompilerParams(use_tc_tiling_on_sc=False),
)
def _sc_pool(tab_hbm, x_hbm, bias_hbm, out_hbm,
             idx_v, idxt_v, rows_v, out_v, bias_v, s0, s1, s2, s3):
    sems = (s0, s1, s2, s3)
    wid = lax.axis_index("s") * NC + lax.axis_index("c")
    base = wid * ROWS_PER_W

    pltpu.sync_copy(bias_hbm, bias_v)
    bias = bias_v[...]

    voffs = list(range(0, HALF - 16, 16)) + [HALF - 16]

    def issue(row_in_chunk, slot):
        for h in range(2):
            for o in voffs:
                v = idx_v[row_in_chunk, h, pl.ds(o, 16)]
                idxt_v[slot, h, pl.ds(o, 16)] = (
                    (v & jnp.int32(SLAB - 1)) << 3) | (v >> 17)
            pltpu.async_copy(tab_hbm.at[idxt_v.at[slot, h]],
                             rows_v.at[slot, h], sems[slot])

    def drain(slot):
        for h in range(2):
            pltpu.make_async_copy(tab_hbm.at[idxt_v.at[slot, h]],
                                  rows_v.at[slot, h], sems[slot]).wait()

    for half in range(2):
        pltpu.sync_copy(x_hbm.at[pl.ds(base + half * CHUNK, CHUNK)], idx_v)
        for q in range(RING):
            issue(q, q)

        def body(r_outer, carry, half=half):
            for q in range(RING):
                row = r_outer * RING + q
                drain(q)
                accs = [rows_v[q, 0, j] for j in range(4)]
                for h in range(2):
                    for j in range(HALF):
                        if h == 0 and j < 4:
                            continue
                        accs[j % 4] = accs[j % 4] + rows_v[q, h, j]
                out_v[half * CHUNK + row] = ((accs[0] + accs[1])
                                             + (accs[2] + accs[3])) + bias

                @pl.when(row + RING < CHUNK)
                def _issue_next(row=row, q=q):
                    issue(row + RING, q)
            return carry

        lax.fori_loop(0, CHUNK // RING, body, None)

    pltpu.sync_copy(out_v, out_hbm.at[pl.ds(base, ROWS_PER_W)])


def kernel(x, emb_table, fc_w, fc_b):
    emb_t = emb_table.T
    w8 = jnp.zeros((NSLAB * EMB, 128), jnp.float32)
    wt = (fc_w.T / SEQ).astype(jnp.float32)
    for s in range(NSLAB):
        w8 = w8.at[s * EMB:(s + 1) * EMB, s * PROJ:s * PROJ + N_CLS].set(wt)
    packed = _pack_table(emb_t, w8)
    tab = packed.reshape(VPAD, PROJ)
    bias_pad = jnp.zeros((PROJ,), jnp.float32).at[:N_CLS].set(fc_b)
    x3 = x.reshape(BATCH, 2, HALF)
    out16 = _sc_pool(tab, x3, bias_pad)
    return out16[:, :N_CLS]
